# baseline (speedup 1.0000x reference)
import functools

import jax
import jax.numpy as jnp
from jax import lax
from jax.experimental import pallas as pl
from jax.experimental.pallas import tpu as pltpu
from jax.experimental.pallas import tpu_sc as plsc

_LANES = 16


@functools.partial(jax.jit, static_argnums=(2, 3, 4, 5))
def _pos_embed_sc(row_embed, col_embed, B, H, W, D):
  info = plsc.get_sparse_core_info()
  NC, NS = info.num_cores, info.num_subcores
  NW = NC * NS
  assert H == NW, "one TEC worker per row index"
  mesh = plsc.VectorSubcoreMesh(core_axis_name="c", subcore_axis_name="s")

  @functools.partial(
      pl.kernel,
      mesh=mesh,
      out_type=jax.ShapeDtypeStruct((B, H * W, 2 * D), jnp.float32),
      scratch_types=[
          pltpu.VMEM((D,), jnp.float32),
          pltpu.VMEM((W, 2 * D), jnp.float32),
          pltpu.SemaphoreType.DMA,
      ],
  )
  def k(row_hbm, col_hbm, out_hbm, row_v, block_v, sem):
    wid = lax.axis_index("s") * NC + lax.axis_index("c")
    a = pltpu.make_async_copy(row_hbm.at[wid], row_v, sem)
    b = pltpu.make_async_copy(
        col_hbm.at[pl.ds(0, W), :], block_v.at[:, pl.ds(D, D)], sem
    )
    a.start()
    b.start()
    a.wait()
    b.wait()
    for c in range(D // _LANES):
      v = row_v[pl.ds(c * _LANES, _LANES)]
      for r in range(W):
        block_v[r, pl.ds(c * _LANES, _LANES)] = v
    outs = [
        pltpu.make_async_copy(
            block_v, out_hbm.at[bb, pl.ds(wid * W, W), :], sem
        )
        for bb in range(B)
    ]
    for cp in outs:
      cp.start()
    for cp in outs:
      cp.wait()

  return k(row_embed, col_embed)


def kernel(x, row_embed, col_embed):
  B, _, H, W = x.shape
  D = row_embed.shape[-1]
  return _pos_embed_sc(row_embed, col_embed, B, H, W, D)
